# Initial kernel scaffold; baseline (speedup 1.0000x reference)
import jax
import jax.numpy as jnp
from jax.experimental import pallas as pl


def kernel(edge_index, src_emb, e_att):
    raise NotImplementedError("write your pallas kernel here")


# speedup vs baseline: 4.4490x; 4.4490x over previous
import functools

import jax
import jax.numpy as jnp
from jax import lax
from jax.experimental import pallas as pl
from jax.experimental.pallas import tpu as pltpu
from jax.experimental.pallas import tpu_sc as plsc

N_NODES = 10000
N_EDGES = 320000
EMB = 128
LANES = 16
VECS_PER_ROW = EMB // LANES

NC = 2
NS = 16
NW = NC * NS
EPW = N_EDGES // NW
B_E = 80
NB = EPW // B_E
N_PAD = 10240
ROWS_PER_TILE = N_PAD // NS
ZCH = 128


def _sc_body(src_hbm, dst_hbm, att_hbm, emb_hbm, out_hbm,
             acc, src_v, dst_v, att_v, rows_v, zbuf, sem):
    cid = lax.axis_index("c")
    sid = lax.axis_index("s")
    wid = sid * NC + cid

    def zrow(i, _):
        for j in range(VECS_PER_ROW):
            zbuf[i, pl.ds(j * LANES, LANES)] = jnp.zeros((LANES,), jnp.float32)
        return 0
    lax.fori_loop(0, ZCH, zrow, 0)
    for k in range(ROWS_PER_TILE // ZCH):
        pltpu.sync_copy(zbuf, acc.at[pl.ds(sid * ROWS_PER_TILE + k * ZCH, ZCH)])
    plsc.subcore_barrier()

    def batch(nb, _):
        base = pl.multiple_of(wid * EPW + nb * B_E, 8)
        pltpu.sync_copy(src_hbm.at[pl.ds(base, B_E)], src_v)
        pltpu.sync_copy(dst_hbm.at[pl.ds(base, B_E)], dst_v)
        pltpu.sync_copy(att_hbm.at[pl.ds(base, B_E)], att_v)
        pltpu.async_copy(emb_hbm.at[src_v], rows_v, sem).wait()

        def scale(c, _):
            att16 = att_v[pl.ds(c * LANES, LANES)]
            for jj in range(LANES):
                sp = lax.gather(
                    att16, jnp.full((LANES, 1), jj, jnp.int32),
                    lax.GatherDimensionNumbers(offset_dims=(),
                                               collapsed_slice_dims=(0,),
                                               start_index_map=(0,)),
                    (1,), mode=lax.GatherScatterMode.PROMISE_IN_BOUNDS)
                i = c * LANES + jj
                for j in range(VECS_PER_ROW):
                    sl = pl.ds(j * LANES, LANES)
                    rows_v[i, sl] = rows_v[i, sl] * sp
            return 0
        lax.fori_loop(0, B_E // LANES, scale, 0)

        pltpu.sync_copy(rows_v, acc.at[dst_v], add=True)
        return 0

    lax.fori_loop(0, NB, batch, 0)
    plsc.subcore_barrier()

    base_r = sid * ROWS_PER_TILE
    pltpu.sync_copy(acc.at[pl.ds(base_r, ROWS_PER_TILE)],
                    out_hbm.at[cid, pl.ds(base_r, ROWS_PER_TILE)])


@jax.jit
def _sc_scatter(src, dst, att, emb):
    mesh = plsc.VectorSubcoreMesh(core_axis_name="c", subcore_axis_name="s")
    return pl.kernel(
        _sc_body,
        out_type=jax.ShapeDtypeStruct((NC, N_PAD, EMB), jnp.float32),
        mesh=mesh,
        scratch_types=[
            pltpu.VMEM_SHARED((N_PAD, EMB), jnp.float32),
            pltpu.VMEM((B_E,), jnp.int32),
            pltpu.VMEM((B_E,), jnp.int32),
            pltpu.VMEM((B_E,), jnp.float32),
            pltpu.VMEM((B_E, EMB), jnp.float32),
            pltpu.VMEM((ZCH, EMB), jnp.float32),
            pltpu.SemaphoreType.DMA,
        ],
    )(src, dst, att, emb)


def _add_body(p_ref, o_ref):
    o_ref[...] = p_ref[0] + p_ref[1]


@jax.jit
def _combine(partial):
    return pl.pallas_call(
        _add_body,
        grid=(10,),
        in_specs=[pl.BlockSpec((NC, 1000, EMB), lambda i: (0, i, 0))],
        out_specs=pl.BlockSpec((1000, EMB), lambda i: (i, 0)),
        out_shape=jax.ShapeDtypeStruct((N_NODES, EMB), jnp.float32),
    )(partial)


def kernel(edge_index, src_emb, e_att):
    src = edge_index[0]
    dst = edge_index[1]
    att = e_att[:, 0]
    partial = _sc_scatter(src, dst, att, src_emb)
    return _combine(partial)


# speedup vs baseline: 9.9263x; 2.2311x over previous
import jax
import jax.numpy as jnp
from jax import lax
from jax.experimental import pallas as pl
from jax.experimental.pallas import tpu as pltpu
from jax.experimental.pallas import tpu_sc as plsc

N_NODES = 10000
N_EDGES = 320000
EMB = 128
LANES = 16
VECS_PER_ROW = EMB // LANES

NC = 2
NS = 16
NW = NC * NS
EPW = N_EDGES // NW
B_E = 80
NB = EPW // B_E
N_PAD = 10112
ROWS_PER_TILE = N_PAD // NS

_SPLAT_DNUMS = lax.GatherDimensionNumbers(
    offset_dims=(), collapsed_slice_dims=(0,), start_index_map=(0,))


def _sc_body(pk_hbm, att_hbm, emb_hbm, out_hbm,
             acc, pk_v, att_v, srcb0, srcb1, dstb0, dstb1, rows0, rows1,
             g0, g1, s0, s1):
    cid = lax.axis_index("c")
    sid = lax.axis_index("s")
    wid = sid * NC + cid

    def zrow(i, _):
        for j in range(VECS_PER_ROW):
            rows1[i, pl.ds(j * LANES, LANES)] = jnp.zeros((LANES,), jnp.float32)
        return 0
    lax.fori_loop(0, B_E, zrow, 0)
    for c in range(B_E // LANES):
        dstb1[pl.ds(c * LANES, LANES)] = jnp.zeros((LANES,), jnp.int32)
    base_r = sid * ROWS_PER_TILE
    for k in range(7):
        pltpu.sync_copy(rows1, acc.at[pl.ds(base_r + k * B_E, B_E)])
    pltpu.sync_copy(rows1.at[pl.ds(0, 72)], acc.at[pl.ds(base_r + 560, 72)])
    plsc.subcore_barrier()

    pltpu.sync_copy(pk_hbm.at[wid], pk_v)
    pltpu.sync_copy(att_hbm.at[wid], att_v)

    def unpack(nb, srcb, dstb):
        for c in range(B_E // LANES):
            sl = pl.ds(c * LANES, LANES)
            pk16 = pk_v[pl.ds(nb * B_E + c * LANES, LANES)]
            srcb[sl] = pk16 & 0xFFFF
            dstb[sl] = pk16 >> 16

    def issue_gather(nb_srcb, buf, sem):
        pltpu.async_copy(emb_hbm.at[nb_srcb], buf, sem)

    def wait_gather(nb_srcb, buf, sem):
        pltpu.make_async_copy(emb_hbm.at[nb_srcb], buf, sem).wait()

    def issue_scatter(buf, dstb, sem):
        pltpu.async_copy(buf, acc.at[dstb], sem, add=True)

    def wait_scatter(buf, dstb, sem):
        pltpu.make_async_copy(buf, acc.at[dstb], sem).wait()

    def scale(nb, buf):
        def chunk(c, _):
            att16 = att_v[pl.ds(nb * B_E + c * LANES, LANES)]
            for jj in range(LANES):
                sp = lax.gather(att16, jnp.full((LANES, 1), jj, jnp.int32),
                                _SPLAT_DNUMS, (1,),
                                mode=lax.GatherScatterMode.PROMISE_IN_BOUNDS)
                i = c * LANES + jj
                for j in range(VECS_PER_ROW):
                    sl = pl.ds(j * LANES, LANES)
                    buf[i, sl] = buf[i, sl] * sp
            return 0
        lax.fori_loop(0, B_E // LANES, chunk, 0)

    issue_scatter(rows1, dstb1, s1)
    unpack(0, srcb0, dstb0)
    issue_gather(srcb0, rows0, g0)

    def pair(g, _):
        t0 = 2 * g
        wait_scatter(rows1, dstb1, s1)
        unpack(t0 + 1, srcb1, dstb1)
        issue_gather(srcb1, rows1, g1)
        wait_gather(srcb0, rows0, g0)
        scale(t0, rows0)
        issue_scatter(rows0, dstb0, s0)
        wait_gather(srcb1, rows1, g1)
        scale(t0 + 1, rows1)
        wait_scatter(rows0, dstb0, s0)
        unpack(t0 + 2, srcb0, dstb0)
        issue_gather(srcb0, rows0, g0)
        issue_scatter(rows1, dstb1, s1)
        return 0
    lax.fori_loop(0, (NB - 1) // 2, pair, 0)

    wait_scatter(rows1, dstb1, s1)
    wait_gather(srcb0, rows0, g0)
    scale(NB - 1, rows0)
    issue_scatter(rows0, dstb0, s0)
    wait_scatter(rows0, dstb0, s0)
    plsc.subcore_barrier()

    pltpu.sync_copy(acc.at[pl.ds(base_r, ROWS_PER_TILE)],
                    out_hbm.at[cid, pl.ds(base_r, ROWS_PER_TILE)])


@jax.jit
def _sc_scatter(pk, att, emb):
    mesh = plsc.VectorSubcoreMesh(core_axis_name="c", subcore_axis_name="s")
    return pl.kernel(
        _sc_body,
        out_type=jax.ShapeDtypeStruct((NC, N_PAD, EMB), jnp.float32),
        mesh=mesh,
        scratch_types=[
            pltpu.VMEM_SHARED((N_PAD, EMB), jnp.float32),
            pltpu.VMEM((EPW,), jnp.int32),
            pltpu.VMEM((EPW,), jnp.float32),
            pltpu.VMEM((B_E,), jnp.int32),
            pltpu.VMEM((B_E,), jnp.int32),
            pltpu.VMEM((B_E,), jnp.int32),
            pltpu.VMEM((B_E,), jnp.int32),
            pltpu.VMEM((B_E, EMB), jnp.float32),
            pltpu.VMEM((B_E, EMB), jnp.float32),
            pltpu.SemaphoreType.DMA,
            pltpu.SemaphoreType.DMA,
            pltpu.SemaphoreType.DMA,
            pltpu.SemaphoreType.DMA,
        ],
    )(pk, att, emb)


def _add_body(p_ref, o_ref):
    o_ref[...] = p_ref[0] + p_ref[1]


@jax.jit
def _combine(partial):
    return pl.pallas_call(
        _add_body,
        grid=(8,),
        in_specs=[pl.BlockSpec((NC, 1264, EMB), lambda i: (0, i, 0))],
        out_specs=pl.BlockSpec((1264, EMB), lambda i: (i, 0)),
        out_shape=jax.ShapeDtypeStruct((N_PAD, EMB), jnp.float32),
    )(partial)


def kernel(edge_index, src_emb, e_att):
    pk = (edge_index[0] | (edge_index[1] << 16)).reshape(NW, EPW)
    att = e_att[:, 0].reshape(NW, EPW)
    partial = _sc_scatter(pk, att, src_emb)
    return _combine(partial)[:N_NODES]


# speedup vs baseline: 11.7251x; 1.1812x over previous
import jax
import jax.numpy as jnp
from jax import lax
from jax.experimental import pallas as pl
from jax.experimental.pallas import tpu as pltpu
from jax.experimental.pallas import tpu_sc as plsc

N_NODES = 10000
N_EDGES = 320000
EMB = 128
LANES = 16
VECS_PER_ROW = EMB // LANES

NC = 2
NS = 16
NW = NC * NS
EPW = N_EDGES // NW
B_E = 80
NB = EPW // B_E
N_PAD = 10112
ROWS_PER_TILE = N_PAD // NS

_SPLAT_DNUMS = lax.GatherDimensionNumbers(
    offset_dims=(), collapsed_slice_dims=(0,), start_index_map=(0,))


def _sc_body(pk_hbm, att_hbm, emb_hbm, out_hbm,
             acc, pk_v,
             srcb0, srcb1, srcb2, dstb0, dstb1, dstb2,
             attb0, attb1, attb2, rows0, rows1, rows2,
             g0, g1, g2, s0, s1, s2):
    cid = lax.axis_index("c")
    sid = lax.axis_index("s")
    wid = sid * NC + cid
    srcb = (srcb0, srcb1, srcb2)
    dstb = (dstb0, dstb1, dstb2)
    attb = (attb0, attb1, attb2)
    rows = (rows0, rows1, rows2)
    gs = (g0, g1, g2)
    ss = (s0, s1, s2)

    def zrow(i, _):
        for j in range(VECS_PER_ROW):
            rows2[i, pl.ds(j * LANES, LANES)] = jnp.zeros((LANES,), jnp.float32)
        return 0
    lax.fori_loop(0, B_E, zrow, 0)
    for c in range(B_E // LANES):
        dstb1[pl.ds(c * LANES, LANES)] = jnp.zeros((LANES,), jnp.int32)
        dstb2[pl.ds(c * LANES, LANES)] = jnp.zeros((LANES,), jnp.int32)
    base_r = sid * ROWS_PER_TILE
    for k in range(7):
        pltpu.async_copy(rows2, acc.at[pl.ds(base_r + k * B_E, B_E)], g1)
    pltpu.async_copy(rows2.at[pl.ds(0, 72)], acc.at[pl.ds(base_r + 560, 72)], g2)
    pltpu.async_copy(pk_hbm.at[wid], pk_v, g0)
    for k in range(7):
        pltpu.make_async_copy(rows2, acc.at[pl.ds(base_r + k * B_E, B_E)], g1).wait()
    pltpu.make_async_copy(rows2.at[pl.ds(0, 72)],
                          acc.at[pl.ds(base_r + 560, 72)], g2).wait()
    pltpu.make_async_copy(pk_hbm.at[wid], pk_v, g0).wait()
    plsc.subcore_barrier()

    def unpack(nb, r):
        for c in range(B_E // LANES):
            sl = pl.ds(c * LANES, LANES)
            pk16 = pk_v[pl.ds(nb * B_E + c * LANES, LANES)]
            srcb[r][sl] = pk16 & 0xFFFF
            dstb[r][sl] = pk16 >> 16

    def att_src(nb):
        return att_hbm.at[pl.ds(pl.multiple_of(wid * EPW + nb * B_E, 8), B_E)]

    def issue_fetch(nb, r):
        unpack(nb, r)
        pltpu.async_copy(emb_hbm.at[srcb[r]], rows[r], gs[r])
        pltpu.async_copy(att_src(nb), attb[r], gs[r])

    def wait_fetch(nb, r):
        pltpu.make_async_copy(emb_hbm.at[srcb[r]], rows[r], gs[r]).wait()
        pltpu.make_async_copy(att_src(nb), attb[r], gs[r]).wait()

    def issue_scatter(r):
        pltpu.async_copy(rows[r], acc.at[dstb[r]], ss[r], add=True)

    def issue_prime(r):
        pltpu.async_copy(rows2, acc.at[dstb[r]], ss[r], add=True)

    def wait_scatter(r):
        pltpu.make_async_copy(rows[r], acc.at[dstb[r]], ss[r]).wait()

    def wait_prime(r):
        pltpu.make_async_copy(rows2, acc.at[dstb[r]], ss[r]).wait()

    def scale(r):
        def chunk(c, _):
            att16 = attb[r][pl.ds(c * LANES, LANES)]
            for jj in range(LANES):
                sp = lax.gather(att16, jnp.full((LANES, 1), jj, jnp.int32),
                                _SPLAT_DNUMS, (1,),
                                mode=lax.GatherScatterMode.PROMISE_IN_BOUNDS)
                i = c * LANES + jj
                for j in range(VECS_PER_ROW):
                    sl = pl.ds(j * LANES, LANES)
                    rows[r][i, sl] = rows[r][i, sl] * sp
            return 0
        lax.fori_loop(0, B_E // LANES, chunk, 0)

    issue_prime(1)
    issue_prime(2)
    issue_fetch(0, 0)

    def step(t, r, rn, first):
        if first:
            wait_prime(rn)
        else:
            wait_scatter(rn)
        issue_fetch(t + 1, rn)
        wait_fetch(t, r)
        scale(r)
        issue_scatter(r)

    step(0, 0, 1, True)
    step(1, 1, 2, True)

    def triple_body(q, _):
        t0 = 3 * q + 2
        step(t0, 2, 0, False)
        step(t0 + 1, 0, 1, False)
        step(t0 + 2, 1, 2, False)
        return 0
    lax.fori_loop(0, 40, triple_body, 0)

    wait_scatter(0)
    issue_fetch(123, 0)
    wait_fetch(122, 2)
    scale(2)
    issue_scatter(2)
    wait_scatter(1)
    issue_fetch(124, 1)
    wait_fetch(123, 0)
    scale(0)
    issue_scatter(0)
    wait_fetch(124, 1)
    scale(1)
    issue_scatter(1)
    wait_scatter(2)
    wait_scatter(0)
    wait_scatter(1)
    plsc.subcore_barrier()

    pltpu.sync_copy(acc.at[pl.ds(base_r, ROWS_PER_TILE)],
                    out_hbm.at[cid, pl.ds(base_r, ROWS_PER_TILE)])


@jax.jit
def _sc_scatter(pk, att, emb):
    mesh = plsc.VectorSubcoreMesh(core_axis_name="c", subcore_axis_name="s")
    return pl.kernel(
        _sc_body,
        out_type=jax.ShapeDtypeStruct((NC, N_PAD, EMB), jnp.float32),
        mesh=mesh,
        scratch_types=[
            pltpu.VMEM_SHARED((N_PAD, EMB), jnp.float32),
            pltpu.VMEM((EPW,), jnp.int32),
            pltpu.VMEM((B_E,), jnp.int32),
            pltpu.VMEM((B_E,), jnp.int32),
            pltpu.VMEM((B_E,), jnp.int32),
            pltpu.VMEM((B_E,), jnp.int32),
            pltpu.VMEM((B_E,), jnp.int32),
            pltpu.VMEM((B_E,), jnp.int32),
            pltpu.VMEM((B_E,), jnp.float32),
            pltpu.VMEM((B_E,), jnp.float32),
            pltpu.VMEM((B_E,), jnp.float32),
            pltpu.VMEM((B_E, EMB), jnp.float32),
            pltpu.VMEM((B_E, EMB), jnp.float32),
            pltpu.VMEM((B_E, EMB), jnp.float32),
            pltpu.SemaphoreType.DMA,
            pltpu.SemaphoreType.DMA,
            pltpu.SemaphoreType.DMA,
            pltpu.SemaphoreType.DMA,
            pltpu.SemaphoreType.DMA,
            pltpu.SemaphoreType.DMA,
        ],
    )(pk, att, emb)


def _add_body(p_ref, o_ref):
    o_ref[...] = p_ref[0] + p_ref[1]


@jax.jit
def _combine(partial):
    return pl.pallas_call(
        _add_body,
        grid=(10,),
        in_specs=[pl.BlockSpec((NC, 1000, EMB), lambda i: (0, i, 0))],
        out_specs=pl.BlockSpec((1000, EMB), lambda i: (i, 0)),
        out_shape=jax.ShapeDtypeStruct((N_NODES, EMB), jnp.float32),
    )(partial)


def kernel(edge_index, src_emb, e_att):
    pk = (edge_index[0] | (edge_index[1] << 16)).reshape(NW, EPW)
    att = e_att[:, 0]
    partial = _sc_scatter(pk, att, src_emb)
    return _combine(partial)
